# baseline (speedup 1.0000x reference)
import functools

import jax
import jax.numpy as jnp
from jax import lax
from jax.experimental import pallas as pl
from jax.experimental.pallas import tpu as pltpu
from jax.experimental.pallas import tpu_sc as plsc

VOCAB = 1000000
EMBED = 256
BATCH = 4096
SEQ = 200

NW = 32
ROWS_PER_W = BATCH // NW
IDX_PER_W = ROWS_PER_W * SEQ
CHUNK = 80
NBUF = 4
ROWS_PER_BLK = 8
CHUNKS_PER_BLK = ROWS_PER_BLK * SEQ // CHUNK
NBLK = ROWS_PER_W // ROWS_PER_BLK
NCHUNKS = IDX_PER_W // CHUNK
NLANE = 16
NVEC = EMBED // NLANE

_mesh = plsc.VectorSubcoreMesh(core_axis_name="c", subcore_axis_name="s")


@functools.partial(
    pl.kernel,
    mesh=_mesh,
    out_type=jax.ShapeDtypeStruct((BATCH, EMBED), jnp.float32),
    scratch_types=(
        [pltpu.VMEM((IDX_PER_W,), jnp.int32)]
        + [pltpu.VMEM((CHUNK, EMBED), jnp.float32)] * NBUF
        + [pltpu.VMEM((ROWS_PER_BLK, EMBED), jnp.float32)]
        + [pltpu.SemaphoreType.DMA] * NBUF
    ),
)
def _pool(idx_hbm, emb_hbm, out_hbm, idx_v, *rest):
    bufs = rest[:NBUF]
    out_v = rest[NBUF]
    sems = rest[NBUF + 1:]
    wid = lax.axis_index("s") * 2 + lax.axis_index("c")
    base = wid * ROWS_PER_W

    pltpu.sync_copy(idx_hbm.at[pl.ds(wid * IDX_PER_W, IDX_PER_W)], idx_v)

    def chunk_src(g):
        return emb_hbm.at[idx_v.at[pl.ds(g * CHUNK, CHUNK)]]

    for k in range(NBUF):
        pltpu.make_async_copy(chunk_src(k), bufs[k], sems[k]).start()

    def accum_range(buf, accs, lo, hi):
        def body(j, a):
            return tuple(a[c] + buf[j, pl.ds(c * NLANE, NLANE)]
                         for c in range(NVEC))
        return plsc.parallel_loop(lo, hi, carry=accs, unroll=4)(body)

    def zeros():
        return tuple(jnp.zeros((NLANE,), jnp.float32) for _ in range(NVEC))

    def emit(slot, accs):
        for c in range(NVEC):
            out_v[slot, pl.ds(c * NLANE, NLANE)] = accs[c] * (1.0 / SEQ)

    def blk_body(blk, carry):
        g0 = blk * CHUNKS_PER_BLK
        acc_a = zeros()
        acc_b = zeros()
        for g2 in range(4):
            for q in range(5):
                k = 5 * g2 + q
                buf, sem = bufs[k % NBUF], sems[k % NBUF]
                pltpu.make_async_copy(chunk_src(g0 + k), buf, sem).wait()
                if q == 0 or q == 1:
                    acc_a = accum_range(buf, acc_a, 0, CHUNK)
                elif q == 2:
                    acc_a = accum_range(buf, acc_a, 0, 40)
                    acc_b = accum_range(buf, acc_b, 40, CHUNK)
                else:
                    acc_b = accum_range(buf, acc_b, 0, CHUNK)

                if k < CHUNKS_PER_BLK - NBUF:
                    pltpu.make_async_copy(chunk_src(g0 + k + NBUF),
                                          buf, sem).start()
                else:
                    @pl.when(blk < NBLK - 1)
                    def _():
                        pltpu.make_async_copy(chunk_src(g0 + k + NBUF),
                                              bufs[k % NBUF],
                                              sems[k % NBUF]).start()
                if q == 2:
                    emit(2 * g2, acc_a)
                elif q == 4:
                    emit(2 * g2 + 1, acc_b)
                    acc_a = zeros()
                    acc_b = zeros()

        off = pl.multiple_of(base + blk * ROWS_PER_BLK, 8)
        pltpu.sync_copy(out_v, out_hbm.at[pl.ds(off, ROWS_PER_BLK)])
        return carry

    lax.fori_loop(0, NBLK, blk_body, 0)


def _linear_body(x_ref, wt_ref, b_ref, o_ref):
    o_ref[...] = (
        jnp.dot(x_ref[...], wt_ref[...], preferred_element_type=jnp.float32)
        + b_ref[...]
    )


_linear = pl.pallas_call(
    _linear_body,
    out_shape=jax.ShapeDtypeStruct((BATCH, 2), jnp.float32),
)


def kernel(inputs, emb, W, b):
    idx_flat = inputs.astype(jnp.int32).reshape(NW * IDX_PER_W)
    pooled = _pool(idx_flat, emb)
    return _linear(pooled, W.T, b.reshape(1, 2))
